# baseline (speedup 1.0000x reference)
import jax
import jax.numpy as jnp
from jax.experimental import pallas as pl
from jax.experimental.pallas import tpu as pltpu
from jax.experimental.pallas import tpu_sc as plsc
from jax.sharding import PartitionSpec as P

K_CODES = 8192
DIM = 32
N_TOK = 8192
TB = 256
NB = N_TOK // TB
BETA_C = 0.25
ENT_RATIO = 0.1
INV_TEMP = 100.0
GW = 128


def _proj_kernel(emb_ref, pw_ref, pb_ref, w_ref, wpad_ref, w2_ref):
    w = jax.lax.dot_general(
        emb_ref[...], pw_ref[...], (((1,), (1,)), ((), ())),
        preferred_element_type=jnp.float32) + pb_ref[...]
    w_ref[...] = w
    wpad_ref[...] = jnp.concatenate(
        [w, jnp.zeros((K_CODES, 128 - DIM), jnp.float32)], axis=1)
    w2_ref[...] = jnp.sum(w * w, axis=1).reshape(1, K_CODES)


def _proj_stage(emb_w, proj_w, proj_b2):
    return pl.pallas_call(
        _proj_kernel,
        out_shape=(
            jax.ShapeDtypeStruct((K_CODES, DIM), jnp.float32),
            jax.ShapeDtypeStruct((K_CODES, 128), jnp.float32),
            jax.ShapeDtypeStruct((1, K_CODES), jnp.float32),
        ),
    )(emb_w, proj_w, proj_b2)


def _vq_kernel(nb, z_ref, w_ref, w2_ref, idx_ref, avgp_ref, acc_ref):
    i = pl.program_id(0)

    @pl.when(i == 0)
    def _init():
        avgp_ref[...] = jnp.zeros((1, K_CODES), jnp.float32)
        acc_ref[0] = 0.0
        acc_ref[1] = 0.0

    z_blk = z_ref[...]
    dot = jax.lax.dot_general(
        z_blk, w_ref[...], (((1,), (1,)), ((), ())),
        preferred_element_type=jnp.float32)
    z2 = jnp.sum(z_blk * z_blk, axis=1, keepdims=True)
    d = z2 + w2_ref[...] - 2.0 * dot

    dmin = jnp.min(d, axis=1)
    iota = jax.lax.broadcasted_iota(jnp.int32, (TB, K_CODES), 1)
    idx = jnp.min(jnp.where(d == dmin[:, None], iota, K_CODES), axis=1)
    idx_ref[0, 0, :] = idx

    acc_ref[0] += jnp.sum(dmin)

    t = (dmin[:, None] - d) * INV_TEMP
    e = jnp.exp(t)
    zsum = jnp.sum(e, axis=1)
    sl = jnp.sum(e * t, axis=1)
    acc_ref[1] += jnp.sum(sl / zsum - jnp.log(zsum))
    avgp_ref[...] += jnp.sum(e / zsum[:, None], axis=0).reshape(1, K_CODES)


def _tc_stage(z_flat, w, w2):
    nb = z_flat.shape[0] // TB
    out_shapes = (
        jax.ShapeDtypeStruct((nb, 1, TB), jnp.int32),
        jax.ShapeDtypeStruct((1, K_CODES), jnp.float32),
        jax.ShapeDtypeStruct((2,), jnp.float32),
    )
    return pl.pallas_call(
        lambda *refs: _vq_kernel(nb, *refs),
        grid=(nb,),
        in_specs=[
            pl.BlockSpec((TB, DIM), lambda i: (i, 0)),
            pl.BlockSpec((K_CODES, DIM), lambda i: (0, 0)),
            pl.BlockSpec((1, K_CODES), lambda i: (0, 0)),
        ],
        out_specs=(
            pl.BlockSpec((1, 1, TB), lambda i: (i, 0, 0)),
            pl.BlockSpec((1, K_CODES), lambda i: (0, 0)),
            pl.BlockSpec(memory_space=pltpu.SMEM),
        ),
        out_shape=out_shapes,
    )(z_flat, w, w2)


def _stats_kernel(avgp_ref, acc_ref, cb_ref, cm_ref, ent_ref):
    inv_n = jnp.float32(1.0 / N_TOK)
    cb = acc_ref[0] * jnp.float32(1.0 / (N_TOK * DIM))
    cb_ref[...] = jnp.full((1, 1), cb, jnp.float32)
    cm_ref[...] = jnp.full((1, 1), BETA_C * cb, jnp.float32)
    ap = avgp_ref[...] * inv_n
    avg_ent = -jnp.sum(ap * jnp.log(ap + 1e-5))
    sample_ent = -(acc_ref[1] * inv_n)
    ent_ref[...] = jnp.full((1, 1), ENT_RATIO * (sample_ent - avg_ent),
                            jnp.float32)


def _stats_stage(avgp, acc):
    return pl.pallas_call(
        _stats_kernel,
        in_specs=[
            pl.BlockSpec((1, K_CODES), lambda: (0, 0)),
            pl.BlockSpec(memory_space=pltpu.SMEM),
        ],
        out_shape=(
            jax.ShapeDtypeStruct((1, 1), jnp.float32),
            jax.ShapeDtypeStruct((1, 1), jnp.float32),
            jax.ShapeDtypeStruct((1, 1), jnp.float32),
        ),
    )(avgp, acc)


def _sc_gather(weight_pad, indices2d):
    n_idx = indices2d.shape[1]
    mesh = plsc.VectorSubcoreMesh(core_axis_name="core",
                                  subcore_axis_name="subcore")

    @pl.kernel(out_type=jax.ShapeDtypeStruct((n_idx, 128), jnp.float32),
               mesh=mesh)
    def kern(w_hbm, i_hbm, o_hbm):
        def body(i_vmem, o_vmem):
            pltpu.sync_copy(w_hbm.at[i_vmem.at[0]], o_vmem)

        pltpu.emit_pipeline(
            body,
            grid=(n_idx // GW,),
            in_specs=[pl.BlockSpec((1, GW), index_map=lambda i: (0, i))],
            out_specs=[pl.BlockSpec((GW, 128), index_map=lambda i: (i, 0))],
            core_axis_name=("core", "subcore"),
            dimension_semantics=(pltpu.PARALLEL,),
        )(i_hbm, o_hbm)

    return kern(weight_pad, indices2d)


def _device_fn(z_loc, emb_w, proj_w, proj_b2, axis_name):
    w, wpad, w2 = _proj_stage(emb_w, proj_w, proj_b2)
    idx, avgp, acc = _tc_stage(z_loc, w, w2)
    if axis_name is not None:
        avgp = jax.lax.psum(avgp, axis_name)
        acc = jax.lax.psum(acc, axis_name)
    cb, cm, ent = _stats_stage(avgp, acc)
    q = _sc_gather(wpad, idx.reshape(1, z_loc.shape[0]))[:, :DIM]
    return q, idx, cb, cm, ent


@jax.jit
def _run(z_flat, emb_w, proj_w, proj_b2):
    n_dev = len(jax.devices())
    nd = 2 if n_dev >= 2 else 1
    if nd == 1:
        return _device_fn(z_flat, emb_w, proj_w, proj_b2, None)
    mesh = jax.make_mesh((nd,), ("dp",))
    in_specs = (P("dp"), P(), P(), P())
    args = tuple(
        jax.reshard(a, jax.sharding.NamedSharding(mesh, s))
        for a, s in zip((z_flat, emb_w, proj_w, proj_b2), in_specs))
    fn = jax.shard_map(
        lambda *a: _device_fn(*a, axis_name="dp"),
        mesh=mesh,
        in_specs=in_specs,
        out_specs=(P("dp"), P("dp"), P(), P(), P()),
        check_vma=False,
    )
    return fn(*args)


def kernel(z, emb_w, proj_w, proj_b):
    b, c, h, w = z.shape
    z_bhwc = jnp.transpose(z, (0, 2, 3, 1))
    z_flat = z_bhwc.reshape(N_TOK, DIM)
    q, idx, cb, cm, ent = _run(z_flat, emb_w, proj_w, proj_b.reshape(1, DIM))
    z_q = jnp.transpose(q.reshape(b, h, w, c), (0, 3, 1, 2))
    flat_indices = idx.reshape(N_TOK)
    usage = jnp.float32(0.0)
    return (z_q, cb[0, 0], cm[0, 0], ent[0, 0], usage, flat_indices)


# speedup vs baseline: 2.7767x; 2.7767x over previous
import jax
import jax.numpy as jnp
from jax.experimental import pallas as pl
from jax.experimental.pallas import tpu as pltpu
from jax.experimental.pallas import tpu_sc as plsc
from jax.sharding import PartitionSpec as P

K_CODES = 8192
DIM = 32
N_TOK = 8192
TB = 256
NB = N_TOK // TB
BETA_C = 0.25
ENT_RATIO = 0.1
INV_TEMP = 100.0
GW = 128


def _proj_kernel(emb_ref, pw_ref, pb_ref, w_ref, wpad_ref, w2_ref):
    w = jax.lax.dot_general(
        emb_ref[...], pw_ref[...], (((1,), (1,)), ((), ())),
        preferred_element_type=jnp.float32) + pb_ref[...]
    w_ref[...] = w
    wpad_ref[...] = jnp.concatenate(
        [w, jnp.zeros((K_CODES, 128 - DIM), jnp.float32)], axis=1)
    w2_ref[...] = jnp.sum(w * w, axis=1).reshape(1, K_CODES)


def _proj_stage(emb_w, proj_w, proj_b2):
    return pl.pallas_call(
        _proj_kernel,
        out_shape=(
            jax.ShapeDtypeStruct((K_CODES, DIM), jnp.float32),
            jax.ShapeDtypeStruct((K_CODES, 128), jnp.float32),
            jax.ShapeDtypeStruct((1, K_CODES), jnp.float32),
        ),
    )(emb_w, proj_w, proj_b2)


def _vq_kernel(nb, z_ref, w_ref, w2_ref, idx_ref, avgp_ref, acc_ref):
    i = pl.program_id(0)

    @pl.when(i == 0)
    def _init():
        avgp_ref[...] = jnp.zeros((1, K_CODES), jnp.float32)
        acc_ref[0] = 0.0
        acc_ref[1] = 0.0

    z_blk = z_ref[...]
    dot = jax.lax.dot_general(
        z_blk, w_ref[...], (((1,), (1,)), ((), ())),
        preferred_element_type=jnp.float32)
    z2 = jnp.sum(z_blk * z_blk, axis=1, keepdims=True)
    d = z2 + w2_ref[...] - 2.0 * dot

    dmin = jnp.min(d, axis=1)
    iota = jax.lax.broadcasted_iota(jnp.int32, (TB, K_CODES), 1)
    idx = jnp.min(jnp.where(d == dmin[:, None], iota, K_CODES), axis=1)
    idx_ref[0, 0, :] = idx

    acc_ref[0] += jnp.sum(dmin)

    t = (dmin[:, None] - d) * INV_TEMP
    e = jnp.exp(t)
    zsum = jnp.sum(e, axis=1)
    sl = jnp.sum(e * t, axis=1)
    acc_ref[1] += jnp.sum(sl / zsum - jnp.log(zsum))
    avgp_ref[...] += jnp.sum(e / zsum[:, None], axis=0).reshape(1, K_CODES)


def _tc_stage(z_flat, w, w2):
    nb = z_flat.shape[0] // TB
    out_shapes = (
        jax.ShapeDtypeStruct((nb, 1, TB), jnp.int32),
        jax.ShapeDtypeStruct((1, K_CODES), jnp.float32),
        jax.ShapeDtypeStruct((2,), jnp.float32),
    )
    return pl.pallas_call(
        lambda *refs: _vq_kernel(nb, *refs),
        grid=(nb,),
        in_specs=[
            pl.BlockSpec((TB, DIM), lambda i: (i, 0)),
            pl.BlockSpec((K_CODES, DIM), lambda i: (0, 0)),
            pl.BlockSpec((1, K_CODES), lambda i: (0, 0)),
        ],
        out_specs=(
            pl.BlockSpec((1, 1, TB), lambda i: (i, 0, 0)),
            pl.BlockSpec((1, K_CODES), lambda i: (0, 0)),
            pl.BlockSpec(memory_space=pltpu.SMEM),
        ),
        out_shape=out_shapes,
    )(z_flat, w, w2)


def _stats_kernel(avgp_ref, acc_ref, cb_ref, cm_ref, ent_ref):
    inv_n = jnp.float32(1.0 / N_TOK)
    cb = acc_ref[0] * jnp.float32(1.0 / (N_TOK * DIM))
    cb_ref[...] = jnp.full((1, 1), cb, jnp.float32)
    cm_ref[...] = jnp.full((1, 1), BETA_C * cb, jnp.float32)
    ap = avgp_ref[...] * inv_n
    avg_ent = -jnp.sum(ap * jnp.log(ap + 1e-5))
    sample_ent = -(acc_ref[1] * inv_n)
    ent_ref[...] = jnp.full((1, 1), ENT_RATIO * (sample_ent - avg_ent),
                            jnp.float32)


def _stats_stage(avgp, acc):
    return pl.pallas_call(
        _stats_kernel,
        in_specs=[
            pl.BlockSpec((1, K_CODES), lambda: (0, 0)),
            pl.BlockSpec(memory_space=pltpu.SMEM),
        ],
        out_shape=(
            jax.ShapeDtypeStruct((1, 1), jnp.float32),
            jax.ShapeDtypeStruct((1, 1), jnp.float32),
            jax.ShapeDtypeStruct((1, 1), jnp.float32),
        ),
    )(avgp, acc)


def _sc_gather(weight_pad, indices2d):
    n_idx = indices2d.shape[1]
    mesh = plsc.VectorSubcoreMesh(core_axis_name="core",
                                  subcore_axis_name="subcore")

    @pl.kernel(out_type=jax.ShapeDtypeStruct((n_idx, 128), jnp.float32),
               mesh=mesh)
    def kern(w_hbm, i_hbm, o_hbm):
        def body(i_vmem, o_vmem):
            pltpu.sync_copy(w_hbm.at[i_vmem.at[0]], o_vmem)

        pltpu.emit_pipeline(
            body,
            grid=(n_idx // GW,),
            in_specs=[pl.BlockSpec((1, GW), index_map=lambda i: (0, i))],
            out_specs=[pl.BlockSpec((GW, 128), index_map=lambda i: (i, 0))],
            core_axis_name=("core", "subcore"),
            dimension_semantics=(pltpu.PARALLEL,),
        )(i_hbm, o_hbm)

    return kern(weight_pad, indices2d)


def _device_fn(z_loc, emb_w, proj_w, proj_b2, axis_name):
    w, wpad, w2 = _proj_stage(emb_w, proj_w, proj_b2)
    idx, avgp, acc = _tc_stage(z_loc, w, w2)
    if axis_name is not None:
        avgp = jax.lax.psum(avgp, axis_name)
        acc = jax.lax.psum(acc, axis_name)
    cb, cm, ent = _stats_stage(avgp, acc)
    q = _sc_gather(wpad, idx.reshape(1, z_loc.shape[0]))[:, :DIM]
    return q, idx, cb, cm, ent


@jax.jit
def _run(z_flat, emb_w, proj_w, proj_b2):
    nd = 1
    if nd == 1:
        return _device_fn(z_flat, emb_w, proj_w, proj_b2, None)
    mesh = jax.make_mesh((nd,), ("dp",))
    in_specs = (P("dp"), P(), P(), P())
    args = tuple(
        jax.reshard(a, jax.sharding.NamedSharding(mesh, s))
        for a, s in zip((z_flat, emb_w, proj_w, proj_b2), in_specs))
    fn = jax.shard_map(
        lambda *a: _device_fn(*a, axis_name="dp"),
        mesh=mesh,
        in_specs=in_specs,
        out_specs=(P("dp"), P("dp"), P(), P(), P()),
        check_vma=False,
    )
    return fn(*args)


def kernel(z, emb_w, proj_w, proj_b):
    b, c, h, w = z.shape
    z_bhwc = jnp.transpose(z, (0, 2, 3, 1))
    z_flat = z_bhwc.reshape(N_TOK, DIM)
    q, idx, cb, cm, ent = _run(z_flat, emb_w, proj_w, proj_b.reshape(1, DIM))
    z_q = jnp.transpose(q.reshape(b, h, w, c), (0, 3, 1, 2))
    flat_indices = idx.reshape(N_TOK)
    usage = jnp.float32(0.0)
    return (z_q, cb[0, 0], cm[0, 0], ent[0, 0], usage, flat_indices)


# speedup vs baseline: 2.8199x; 1.0156x over previous
import jax
import jax.numpy as jnp
from jax.experimental import pallas as pl
from jax.experimental.pallas import tpu as pltpu
from jax.experimental.pallas import tpu_sc as plsc
from jax.sharding import PartitionSpec as P

K_CODES = 8192
DIM = 32
N_TOK = 8192
TB = 512
NB = N_TOK // TB
BETA_C = 0.25
ENT_RATIO = 0.1
INV_TEMP = 100.0
INV_TEMP_LOG2 = 144.26950408889634
LN2 = 0.6931471805599453
GW = 128


def _proj_kernel(emb_ref, pw_ref, pb_ref, wm2_ref, wpad_ref, w2_ref):
    w = jax.lax.dot_general(
        emb_ref[...], pw_ref[...], (((1,), (1,)), ((), ())),
        preferred_element_type=jnp.float32) + pb_ref[...]
    wm2_ref[...] = -2.0 * w
    wpad_ref[...] = jnp.concatenate(
        [w, jnp.zeros((K_CODES, 128 - DIM), jnp.float32)], axis=1)
    w2_ref[...] = jnp.sum(w * w, axis=1).reshape(1, K_CODES)


def _proj_stage(emb_w, proj_w, proj_b2):
    return pl.pallas_call(
        _proj_kernel,
        out_shape=(
            jax.ShapeDtypeStruct((K_CODES, DIM), jnp.float32),
            jax.ShapeDtypeStruct((K_CODES, 128), jnp.float32),
            jax.ShapeDtypeStruct((1, K_CODES), jnp.float32),
        ),
    )(emb_w, proj_w, proj_b2)


def _vq_kernel(nb, z_ref, w_ref, w2_ref, idx_ref, avgp_ref, acc_ref):
    i = pl.program_id(0)

    @pl.when(i == 0)
    def _init():
        avgp_ref[...] = jnp.zeros((1, K_CODES), jnp.float32)
        acc_ref[0] = 0.0
        acc_ref[1] = 0.0

    z_blk = z_ref[...]
    dotm2 = jax.lax.dot_general(
        z_blk, w_ref[...], (((1,), (1,)), ((), ())),
        preferred_element_type=jnp.float32)
    z2 = jnp.sum(z_blk * z_blk, axis=1, keepdims=True)
    d = z2 + w2_ref[...] + dotm2

    dmin = jnp.min(d, axis=1)

    acc_ref[0] += jnp.sum(dmin)

    t2f = (dmin[:, None] - d) * INV_TEMP_LOG2
    iota = jax.lax.broadcasted_iota(jnp.int32, (TB, K_CODES), 1)
    idx = jnp.min(jnp.where(t2f == 0.0, iota, K_CODES), axis=1)
    idx_ref[0, 0, :] = idx

    e = jnp.exp2(t2f)
    zsum = jnp.sum(e, axis=1)
    s2 = jnp.sum(e * t2f, axis=1)
    acc_ref[1] += jnp.sum(LN2 * (s2 / zsum) - jnp.log(zsum))
    avgp_ref[...] += jnp.sum(e / zsum[:, None], axis=0).reshape(1, K_CODES)


def _tc_stage(z_flat, w, w2):
    nb = z_flat.shape[0] // TB
    out_shapes = (
        jax.ShapeDtypeStruct((nb, 1, TB), jnp.int32),
        jax.ShapeDtypeStruct((1, K_CODES), jnp.float32),
        jax.ShapeDtypeStruct((2,), jnp.float32),
    )
    return pl.pallas_call(
        lambda *refs: _vq_kernel(nb, *refs),
        grid=(nb,),
        in_specs=[
            pl.BlockSpec((TB, DIM), lambda i: (i, 0)),
            pl.BlockSpec((K_CODES, DIM), lambda i: (0, 0)),
            pl.BlockSpec((1, K_CODES), lambda i: (0, 0)),
        ],
        out_specs=(
            pl.BlockSpec((1, 1, TB), lambda i: (i, 0, 0)),
            pl.BlockSpec((1, K_CODES), lambda i: (0, 0)),
            pl.BlockSpec(memory_space=pltpu.SMEM),
        ),
        out_shape=out_shapes,
    )(z_flat, w, w2)


def _stats_kernel(avgp_ref, acc_ref, cb_ref, cm_ref, ent_ref):
    inv_n = jnp.float32(1.0 / N_TOK)
    cb = acc_ref[0] * jnp.float32(1.0 / (N_TOK * DIM))
    cb_ref[...] = jnp.full((1, 1), cb, jnp.float32)
    cm_ref[...] = jnp.full((1, 1), BETA_C * cb, jnp.float32)
    ap = avgp_ref[...] * inv_n
    avg_ent = -jnp.sum(ap * jnp.log(ap + 1e-5))
    sample_ent = -(acc_ref[1] * inv_n)
    ent_ref[...] = jnp.full((1, 1), ENT_RATIO * (sample_ent - avg_ent),
                            jnp.float32)


def _stats_stage(avgp, acc):
    return pl.pallas_call(
        _stats_kernel,
        in_specs=[
            pl.BlockSpec((1, K_CODES), lambda: (0, 0)),
            pl.BlockSpec(memory_space=pltpu.SMEM),
        ],
        out_shape=(
            jax.ShapeDtypeStruct((1, 1), jnp.float32),
            jax.ShapeDtypeStruct((1, 1), jnp.float32),
            jax.ShapeDtypeStruct((1, 1), jnp.float32),
        ),
    )(avgp, acc)


def _sc_gather(weight_pad, indices2d):
    n_idx = indices2d.shape[1]
    mesh = plsc.VectorSubcoreMesh(core_axis_name="core",
                                  subcore_axis_name="subcore")

    @pl.kernel(out_type=jax.ShapeDtypeStruct((n_idx, 128), jnp.float32),
               mesh=mesh)
    def kern(w_hbm, i_hbm, o_hbm):
        def body(i_vmem, o_vmem):
            pltpu.sync_copy(w_hbm.at[i_vmem.at[0]], o_vmem)

        pltpu.emit_pipeline(
            body,
            grid=(n_idx // GW,),
            in_specs=[pl.BlockSpec((1, GW), index_map=lambda i: (0, i))],
            out_specs=[pl.BlockSpec((GW, 128), index_map=lambda i: (i, 0))],
            core_axis_name=("core", "subcore"),
            dimension_semantics=(pltpu.PARALLEL,),
        )(i_hbm, o_hbm)

    return kern(weight_pad, indices2d)


def _device_fn(z_loc, emb_w, proj_w, proj_b2, axis_name):
    w, wpad, w2 = _proj_stage(emb_w, proj_w, proj_b2)
    idx, avgp, acc = _tc_stage(z_loc, w, w2)
    if axis_name is not None:
        avgp = jax.lax.psum(avgp, axis_name)
        acc = jax.lax.psum(acc, axis_name)
    cb, cm, ent = _stats_stage(avgp, acc)
    q = _sc_gather(wpad, idx.reshape(1, z_loc.shape[0]))[:, :DIM]
    return q, idx, cb, cm, ent


@jax.jit
def _run(z_flat, emb_w, proj_w, proj_b2):
    nd = 1
    if nd == 1:
        return _device_fn(z_flat, emb_w, proj_w, proj_b2, None)
    mesh = jax.make_mesh((nd,), ("dp",))
    in_specs = (P("dp"), P(), P(), P())
    args = tuple(
        jax.reshard(a, jax.sharding.NamedSharding(mesh, s))
        for a, s in zip((z_flat, emb_w, proj_w, proj_b2), in_specs))
    fn = jax.shard_map(
        lambda *a: _device_fn(*a, axis_name="dp"),
        mesh=mesh,
        in_specs=in_specs,
        out_specs=(P("dp"), P("dp"), P(), P(), P()),
        check_vma=False,
    )
    return fn(*args)


def kernel(z, emb_w, proj_w, proj_b):
    b, c, h, w = z.shape
    z_bhwc = jnp.transpose(z, (0, 2, 3, 1))
    z_flat = z_bhwc.reshape(N_TOK, DIM)
    q, idx, cb, cm, ent = _run(z_flat, emb_w, proj_w, proj_b.reshape(1, DIM))
    z_q = jnp.transpose(q.reshape(b, h, w, c), (0, 3, 1, 2))
    flat_indices = idx.reshape(N_TOK)
    usage = jnp.float32(0.0)
    return (z_q, cb[0, 0], cm[0, 0], ent[0, 0], usage, flat_indices)


# speedup vs baseline: 3.1222x; 1.1072x over previous
import jax
import jax.numpy as jnp
from jax.experimental import pallas as pl
from jax.experimental.pallas import tpu as pltpu
from jax.experimental.pallas import tpu_sc as plsc

K_CODES = 8192
DIM = 32
N_TOK = 8192
TB = 512
NB = N_TOK // TB
BETA_C = 0.25
ENT_RATIO = 0.1
INV_TEMP_LOG2 = 144.26950408889634
LN2 = 0.6931471805599453
GW = 128


def _proj_kernel(emb_ref, pw_ref, pb_ref, wm2_ref, wpad_ref, w2_ref):
    w = jax.lax.dot_general(
        emb_ref[...], pw_ref[...], (((1,), (1,)), ((), ())),
        preferred_element_type=jnp.float32) + pb_ref[...]
    wm2_ref[...] = -2.0 * w
    wpad_ref[...] = jnp.concatenate(
        [w, jnp.zeros((K_CODES, 128 - DIM), jnp.float32)], axis=1)
    w2_ref[...] = jnp.sum(w * w, axis=1).reshape(1, K_CODES)


def _proj_stage(emb_w, proj_w, proj_b2):
    return pl.pallas_call(
        _proj_kernel,
        out_shape=(
            jax.ShapeDtypeStruct((K_CODES, DIM), jnp.float32),
            jax.ShapeDtypeStruct((K_CODES, 128), jnp.float32),
            jax.ShapeDtypeStruct((1, K_CODES), jnp.float32),
        ),
    )(emb_w, proj_w, proj_b2)


def _vq_kernel(nb, z_ref, w_ref, w2_ref, idx_ref, avgp_ref, acc_ref):
    i = pl.program_id(0)

    @pl.when(i == 0)
    def _init():
        avgp_ref[...] = jnp.zeros((1, K_CODES), jnp.float32)
        acc_ref[0] = 0.0
        acc_ref[1] = 0.0

    z_blk = z_ref[...]
    dotm2 = jax.lax.dot_general(
        z_blk, w_ref[...], (((1,), (1,)), ((), ())),
        preferred_element_type=jnp.float32)
    z2 = jnp.sum(z_blk * z_blk, axis=1, keepdims=True)
    d = z2 + w2_ref[...] + dotm2

    dmin = jnp.min(d, axis=1)

    acc_ref[0] += jnp.sum(dmin)

    idx = jnp.argmin(d, axis=1).astype(jnp.int32)
    idx_ref[0, 0, :] = idx

    t2f = (dmin[:, None] - d) * INV_TEMP_LOG2

    e = jnp.exp2(t2f)
    zsum = jnp.sum(e, axis=1)
    s2 = jnp.sum(e * t2f, axis=1)
    acc_ref[1] += jnp.sum(LN2 * (s2 / zsum) - jnp.log(zsum))
    avgp_ref[...] += jnp.sum(e / zsum[:, None], axis=0).reshape(1, K_CODES)


def _tc_stage(z_flat, w, w2):
    nb = z_flat.shape[0] // TB
    out_shapes = (
        jax.ShapeDtypeStruct((nb, 1, TB), jnp.int32),
        jax.ShapeDtypeStruct((1, K_CODES), jnp.float32),
        jax.ShapeDtypeStruct((2,), jnp.float32),
    )
    return pl.pallas_call(
        lambda *refs: _vq_kernel(nb, *refs),
        grid=(nb,),
        in_specs=[
            pl.BlockSpec((TB, DIM), lambda i: (i, 0)),
            pl.BlockSpec((K_CODES, DIM), lambda i: (0, 0)),
            pl.BlockSpec((1, K_CODES), lambda i: (0, 0)),
        ],
        out_specs=(
            pl.BlockSpec((1, 1, TB), lambda i: (i, 0, 0)),
            pl.BlockSpec((1, K_CODES), lambda i: (0, 0)),
            pl.BlockSpec(memory_space=pltpu.SMEM),
        ),
        out_shape=out_shapes,
    )(z_flat, w, w2)


def _stats_kernel(avgp_ref, acc_ref, cb_ref, cm_ref, ent_ref):
    inv_n = jnp.float32(1.0 / N_TOK)
    cb = acc_ref[0] * jnp.float32(1.0 / (N_TOK * DIM))
    cb_ref[...] = jnp.full((1, 1), cb, jnp.float32)
    cm_ref[...] = jnp.full((1, 1), BETA_C * cb, jnp.float32)
    ap = avgp_ref[...] * inv_n
    avg_ent = -jnp.sum(ap * jnp.log(ap + 1e-5))
    sample_ent = -(acc_ref[1] * inv_n)
    ent_ref[...] = jnp.full((1, 1), ENT_RATIO * (sample_ent - avg_ent),
                            jnp.float32)


def _stats_stage(avgp, acc):
    return pl.pallas_call(
        _stats_kernel,
        in_specs=[
            pl.BlockSpec((1, K_CODES), lambda: (0, 0)),
            pl.BlockSpec(memory_space=pltpu.SMEM),
        ],
        out_shape=(
            jax.ShapeDtypeStruct((1, 1), jnp.float32),
            jax.ShapeDtypeStruct((1, 1), jnp.float32),
            jax.ShapeDtypeStruct((1, 1), jnp.float32),
        ),
    )(avgp, acc)


def _sc_gather(weight_pad, indices2d):
    n_idx = indices2d.shape[1]
    mesh = plsc.VectorSubcoreMesh(core_axis_name="core",
                                  subcore_axis_name="subcore")

    @pl.kernel(out_type=jax.ShapeDtypeStruct((n_idx, 128), jnp.float32),
               mesh=mesh)
    def kern(w_hbm, i_hbm, o_hbm):
        def body(i_vmem, o_vmem):
            pltpu.sync_copy(w_hbm.at[i_vmem.at[0]], o_vmem)

        pltpu.emit_pipeline(
            body,
            grid=(n_idx // GW,),
            in_specs=[pl.BlockSpec((1, GW), index_map=lambda i: (0, i))],
            out_specs=[pl.BlockSpec((GW, 128), index_map=lambda i: (i, 0))],
            core_axis_name=("core", "subcore"),
            dimension_semantics=(pltpu.PARALLEL,),
        )(i_hbm, o_hbm)

    return kern(weight_pad, indices2d)


@jax.jit
def _run(z_flat, emb_w, proj_w, proj_b2):
    w, wpad, w2 = _proj_stage(emb_w, proj_w, proj_b2)
    idx, avgp, acc = _tc_stage(z_flat, w, w2)
    cb, cm, ent = _stats_stage(avgp, acc)
    q = _sc_gather(wpad, idx.reshape(1, N_TOK))[:, :DIM]
    return q, idx, cb, cm, ent


def kernel(z, emb_w, proj_w, proj_b):
    b, c, h, w = z.shape
    z_bhwc = jnp.transpose(z, (0, 2, 3, 1))
    z_flat = z_bhwc.reshape(N_TOK, DIM)
    q, idx, cb, cm, ent = _run(z_flat, emb_w, proj_w, proj_b.reshape(1, DIM))
    z_q = jnp.transpose(q.reshape(b, h, w, c), (0, 3, 1, 2))
    flat_indices = idx.reshape(N_TOK)
    usage = jnp.float32(0.0)
    return (z_q, cb[0, 0], cm[0, 0], ent[0, 0], usage, flat_indices)
